# Initial kernel scaffold; baseline (speedup 1.0000x reference)
import jax
import jax.numpy as jnp
from jax.experimental import pallas as pl


def kernel(word, table):
    raise NotImplementedError("write your pallas kernel here")


# speedup vs baseline: 1.4933x; 1.4933x over previous
import functools

import jax
import jax.numpy as jnp
from jax import lax
from jax.experimental import pallas as pl
from jax.experimental.pallas import tpu as pltpu
from jax.experimental.pallas import tpu_sc as plsc

_NUM_CORES = 2
_NUM_SUBCORES = 16
_NW = _NUM_CORES * _NUM_SUBCORES

_CHUNK = 128
_NBUF = 4


@functools.lru_cache(maxsize=None)
def _build(B, D, n_chunks):
  b_per_w = B // _NW
  mesh = plsc.VectorSubcoreMesh(
      core_axis_name="c", subcore_axis_name="s",
      num_cores=_NUM_CORES, num_subcores=_NUM_SUBCORES)

  @functools.partial(
      pl.kernel,
      out_type=jax.ShapeDtypeStruct((B, D), jnp.float32),
      mesh=mesh,
      scratch_types=[
          pltpu.VMEM((n_chunks, _CHUNK), jnp.int32),
          pltpu.VMEM((_NBUF, _CHUNK, D), jnp.float32),
          pltpu.SemaphoreType.DMA((_NBUF,)),
      ],
      compiler_params=pltpu.CompilerParams(use_tc_tiling_on_sc=False),
  )
  def k(word_hbm, table_hbm, out_hbm, idx_v, rows_v, gsem):
    wid = lax.axis_index("s") * _NUM_CORES + lax.axis_index("c")
    base = wid * b_per_w
    pltpu.sync_copy(word_hbm.at[wid], idx_v)

    def gather_start(j, b):
      pltpu.make_async_copy(
          table_hbm.at[idx_v.at[j]], rows_v.at[b], gsem.at[b]).start()

    def gather_wait_and_store(j, b):
      pltpu.make_async_copy(
          table_hbm.at[idx_v.at[j]], rows_v.at[b], gsem.at[b]).wait()
      pltpu.sync_copy(rows_v.at[b],
                      out_hbm.at[pl.ds(base + j * _CHUNK, _CHUNK)])

    for b in range(_NBUF):
      gather_start(b, b)

    @pl.loop(0, n_chunks - _NBUF, step=_NBUF)
    def _(g):
      for b in range(_NBUF):
        gather_wait_and_store(g + b, b)
        gather_start(g + b + _NBUF, b)

    for b in range(_NBUF):
      gather_wait_and_store(n_chunks - _NBUF + b, b)

  return k


def kernel(word, table):
  batch, hist = word.shape
  total = batch * hist
  assert total % (_NW * _CHUNK) == 0
  n_chunks = total // (_NW * _CHUNK)
  idx = word.astype(jnp.int32).reshape(_NW, n_chunks, _CHUNK)
  out = _build(total, table.shape[1], n_chunks)(idx, table)
  return out.reshape(batch, hist, table.shape[1])
